# baseline (speedup 1.0000x reference)
import functools

import jax
import jax.numpy as jnp
from jax import lax
from jax.experimental import pallas as pl
from jax.experimental.pallas import tpu as pltpu
from jax.experimental.pallas import tpu_sc as plsc

_B, _N, _K, _C, _D = 2, 512, 16, 512, 256
_NC, _NS = 2, 16
_NW = _NC * _NS
_TI = 128
_CH = 128
_TW = 3 * _D // 2

_F32 = jnp.float32
_C11 = (((1,), (1,)), ((), ()))


def _pack_bf16_pair(a, b):
    ua = lax.bitcast_convert_type(a, jnp.uint32)
    ub = lax.bitcast_convert_type(b, jnp.uint32)
    ua = ua + (jnp.uint32(0x7FFF) + ((ua >> 16) & jnp.uint32(1)))
    ub = ub + (jnp.uint32(0x7FFF) + ((ub >> 16) & jnp.uint32(1)))
    packed = (ua & jnp.uint32(0xFFFF0000)) | (ub >> 16)
    return lax.bitcast_convert_type(packed, _F32)


def _unpack_bf16_pair(p):
    up = lax.bitcast_convert_type(p, jnp.uint32)
    a = lax.bitcast_convert_type(up & jnp.uint32(0xFFFF0000), _F32)
    b = lax.bitcast_convert_type(up << 16, _F32)
    return a, b


def _stage_a1_body(f_ref, Wfc1_ref, bfc1_ref, Wfb1_ref, Wks_ref, Wvs_ref,
                   idx_ref, T_ref, x_ref):
    f = f_ref[0]
    m = lax.dot_general(f, f, (((0,), (0,)), ((), ())))
    xx = jnp.sum(f * f, axis=0, keepdims=True)
    inner = -2.0 * m
    p = (-xx) - inner
    iota = lax.broadcasted_iota(jnp.int32, (1, _N), 1)
    cols = []
    for _ in range(_K):
        mx = jnp.max(p, axis=1, keepdims=True)
        sel = jnp.min(jnp.where(p == mx, iota, _N), axis=1, keepdims=True)
        cols.append(sel)
        p = jnp.where(iota == sel, -jnp.inf, p)
    idx = jnp.concatenate(cols, axis=1)
    idx_ref[0] = idx + pl.program_id(0) * _N

    Wfb1 = Wfb1_ref[...]
    Wa = Wfb1[:, :_C]
    cN1 = (((0,), (1,)), ((), ()))
    x = lax.dot_general(f, Wfc1_ref[...], cN1) + bfc1_ref[...]
    pa = lax.dot_general(f, Wa, cN1)
    k_ = lax.dot_general(x, Wks_ref[...], _C11)
    v = lax.dot_general(x, Wvs_ref[...], _C11)
    T = jnp.concatenate([pa, k_, v], axis=1)
    T_ref[0] = _pack_bf16_pair(T[:, :_TW], T[:, _TW:])
    x_ref[0] = x


def _stage_a1(features, W_fc1, b_fc1, W_fb1, W_ks, W_vs, interpret=False):
    def wspec(shape):
        return pl.BlockSpec(shape, lambda b: (0,) * len(shape))

    return pl.pallas_call(
        _stage_a1_body,
        grid=(_B,),
        in_specs=[
            pl.BlockSpec((1, _C, _N), lambda b: (b, 0, 0)),
            wspec((_D, _C)), wspec((1, _D)), wspec((_D, 2 * _C)),
            wspec((_D, _D)), wspec((_D, _D)),
        ],
        out_specs=[
            pl.BlockSpec((1, _N, _K), lambda b: (b, 0, 0)),
            pl.BlockSpec((1, _N, _TW), lambda b: (b, 0, 0)),
            pl.BlockSpec((1, _N, _D), lambda b: (b, 0, 0)),
        ],
        out_shape=[
            jax.ShapeDtypeStruct((_B, _N, _K), jnp.int32),
            jax.ShapeDtypeStruct((_B, _N, _TW), _F32),
            jax.ShapeDtypeStruct((_B, _N, _D), _F32),
        ],
        interpret=interpret,
    )(features, W_fc1, b_fc1.reshape(1, _D), W_fb1, W_ks, W_vs)


def _stage_a2_body(f_ref, x_ref, Wfb1_ref, Wqs_ref, q_ref, pd_ref):
    f = f_ref[0]
    x = x_ref[0]
    Wfb1 = Wfb1_ref[...]
    Wd = Wfb1[:, _C:] - Wfb1[:, :_C]
    cN1 = (((0,), (1,)), ((), ()))
    pd_ref[0] = lax.dot_general(f, Wd, cN1)
    q_ref[0] = lax.dot_general(x, Wqs_ref[...], _C11)


def _stage_a2(features, x, W_fb1, W_qs, interpret=False):
    def wspec(shape):
        return pl.BlockSpec(shape, lambda b: (0,) * len(shape))

    return pl.pallas_call(
        _stage_a2_body,
        grid=(_B,),
        in_specs=[
            pl.BlockSpec((1, _C, _N), lambda b: (b, 0, 0)),
            pl.BlockSpec((1, _N, _D), lambda b: (b, 0, 0)),
            wspec((_D, 2 * _C)), wspec((_D, _D)),
        ],
        out_specs=[
            pl.BlockSpec((1, _N, _D), lambda b: (b, 0, 0)),
            pl.BlockSpec((1, _N, _D), lambda b: (b, 0, 0)),
        ],
        out_shape=[
            jax.ShapeDtypeStruct((_B, _N, _D), _F32),
            jax.ShapeDtypeStruct((_B, _N, _D), _F32),
        ],
        interpret=interpret,
    )(features, x, W_fb1, W_qs)


def _sc_gather(table, idx2d):
    nrows = idx2d.shape[0] * idx2d.shape[1]
    per_w = nrows // _NW
    nch = per_w // _CH
    mesh = plsc.VectorSubcoreMesh(core_axis_name="c", subcore_axis_name="s")

    @functools.partial(
        pl.kernel, mesh=mesh,
        out_type=jax.ShapeDtypeStruct((nrows, _TW), _F32),
        scratch_types=[
            pltpu.VMEM((nch, _CH), jnp.int32),
            pltpu.VMEM((2, _CH, _TW), _F32),
            pltpu.SemaphoreType.DMA,
            pltpu.SemaphoreType.DMA,
        ],
    )
    def gk(table_hbm, idx_hbm, out_hbm, idx_v, rows_v, sem0, sem1):
        wid = lax.axis_index("s") * _NC + lax.axis_index("c")
        base = wid * per_w
        pltpu.sync_copy(idx_hbm.at[pl.ds(wid * nch, nch)], idx_v)
        sems = [sem0, sem1]
        cps = [None, None]
        cps[0] = pltpu.async_copy(table_hbm.at[idx_v.at[0]], rows_v.at[0],
                                  sem0)
        for c in range(nch):
            p = c & 1
            if c + 1 < nch:
                pn = (c + 1) & 1
                cps[pn] = pltpu.async_copy(table_hbm.at[idx_v.at[c + 1]],
                                           rows_v.at[pn], sems[pn])
            cps[p].wait()
            pltpu.sync_copy(rows_v.at[p], out_hbm.at[pl.ds(base + c * _CH,
                                                           _CH)])

    return gk(table, idx2d)


def _tail(res1, xr, Wfc2, bfc2, bng, bnb, Wqk, Wv, bv, Wt, bt, abng, abnb):
    y = lax.dot_general(res1, Wfc2, _C11) + bfc2
    m1 = jnp.mean(y, axis=0, keepdims=True)
    v1 = jnp.mean((y - m1) ** 2, axis=0, keepdims=True)
    res = jnp.maximum(bng * (y - m1) * lax.rsqrt(v1 + 1e-5) + bnb, 0.0) + xr
    trs = []
    for b in range(_B):
        rb = res[b * _N:(b + 1) * _N]
        xq = lax.dot_general(rb, Wqk, _C11)
        e = lax.dot_general(xq, xq, _C11)
        ee = jnp.exp(e - jnp.max(e, axis=1, keepdims=True))
        att = ee / jnp.sum(ee, axis=1, keepdims=True)
        att = att / (1e-9 + jnp.sum(att, axis=0, keepdims=True))
        xv = lax.dot_general(rb, Wv, _C11) + bv
        x_r = lax.dot_general(att, xv, (((0,), (0,)), ((), ())))
        trs.append(lax.dot_general(rb - x_r, Wt, _C11) + bt)
    tr = jnp.concatenate(trs, axis=0)
    m2 = jnp.mean(tr, axis=0, keepdims=True)
    v2 = jnp.mean((tr - m2) ** 2, axis=0, keepdims=True)
    xr2 = jnp.maximum(abng * (tr - m2) * lax.rsqrt(v2 + 1e-5) + abnb, 0.0)
    return res + xr2


def _stage_cd_body(G_ref, pd_ref, q_ref, x_ref, Wfb2_ref, bfb1_ref,
                   bfb2_ref, Wg1_ref, bg1_ref, Wg2_ref, bg2_ref, Wfc2_ref,
                   bfc2_ref, bng_ref, bnb_ref, Wqk_ref, Wv_ref, bv_ref,
                   Wt_ref, bt_ref, abng_ref, abnb_ref, attn_ref, out_ref,
                   acc_ref):
    p = pl.program_id(0)
    ghi, glo = _unpack_bf16_pair(G_ref[...])
    G = jnp.concatenate([ghi, glo], axis=1)
    pd = pd_ref[...]
    q = q_ref[...]
    pd_b = jnp.broadcast_to(pd[:, None, :], (_TI, _K, _D)).reshape(_TI * _K,
                                                                   _D)
    q_b = jnp.broadcast_to(q[:, None, :], (_TI, _K, _D)).reshape(_TI * _K, _D)
    h1 = jnp.maximum(G[:, :_D] + pd_b + bfb1_ref[...], 0.0)
    kf = lax.dot_general(h1, Wfb2_ref[...], _C11) + bfb2_ref[...]
    t = q_b - G[:, _D:2 * _D] + kf
    g1 = jnp.maximum(lax.dot_general(t, Wg1_ref[...], _C11) + bg1_ref[...],
                     0.0)
    araw = lax.dot_general(g1, Wg2_ref[...], _C11) + bg2_ref[...]
    s = (araw * (1.0 / 16.0)).reshape(_TI, _K, _D)
    mx = jnp.max(s, axis=1, keepdims=True)
    sh = s - mx
    attn = sh - jnp.log(jnp.sum(jnp.exp(sh), axis=1, keepdims=True))
    attn_ref[...] = attn.reshape(1, _TI, _K, _D)
    vkf = (G[:, 2 * _D:] + kf).reshape(_TI, _K, _D)
    acc_ref[pl.ds(p * _TI, _TI), :] = jnp.sum(attn * vkf, axis=1)

    nt = (_B * _N) // _TI

    @pl.when(p == nt - 1)
    def _():
        out_ref[...] = _tail(
            acc_ref[...], x_ref[...], Wfc2_ref[...], bfc2_ref[...],
            bng_ref[...], bnb_ref[...], Wqk_ref[...], Wv_ref[...],
            bv_ref[...], Wt_ref[...], bt_ref[...], abng_ref[...],
            abnb_ref[...]).reshape(_B, _N, _D)


def _stage_cd(G, pd, q, x, W_fb2, b_fb1, b_fb2, W_g1, b_g1, W_g2, b_g2,
              W_fc2, b_fc2, bn_g, bn_b, W_qk, W_v, b_v, W_t, b_t, abn_g,
              abn_b, interpret=False):
    nt = (_B * _N) // _TI

    def wspec(shape):
        return pl.BlockSpec(shape, lambda p: (0,) * len(shape))

    return pl.pallas_call(
        _stage_cd_body,
        grid=(nt,),
        in_specs=[
            pl.BlockSpec((_TI * _K, _TW), lambda p: (p, 0)),
            pl.BlockSpec((_TI, _D), lambda p: (p, 0)),
            pl.BlockSpec((_TI, _D), lambda p: (p, 0)),
            wspec((_B * _N, _D)),
            wspec((_D, _D)), wspec((1, _D)), wspec((1, _D)),
            wspec((_D, _D)), wspec((1, _D)),
            wspec((_D, _D)), wspec((1, _D)),
            wspec((_D, _D)), wspec((1, _D)), wspec((1, _D)), wspec((1, _D)),
            wspec((_D // 4, _D)),
            wspec((_D, _D)), wspec((1, _D)),
            wspec((_D, _D)), wspec((1, _D)), wspec((1, _D)), wspec((1, _D)),
        ],
        out_specs=[
            pl.BlockSpec((1, _TI, _K, _D),
                         lambda p: (p // (_N // _TI), p % (_N // _TI), 0, 0)),
            wspec((_B, _N, _D)),
        ],
        out_shape=[
            jax.ShapeDtypeStruct((_B, _N, _K, _D), _F32),
            jax.ShapeDtypeStruct((_B, _N, _D), _F32),
        ],
        scratch_shapes=[pltpu.VMEM((_B * _N, _D), _F32)],
        interpret=interpret,
    )(G, pd, q, x, W_fb2, b_fb1.reshape(1, _D), b_fb2.reshape(1, _D),
      W_g1, b_g1.reshape(1, _D), W_g2, b_g2.reshape(1, _D),
      W_fc2, b_fc2.reshape(1, _D), bn_g.reshape(1, _D), bn_b.reshape(1, _D),
      W_qk, W_v, b_v.reshape(1, _D), W_t, b_t.reshape(1, _D),
      abn_g.reshape(1, _D), abn_b.reshape(1, _D))


def kernel(features, W_fc1, b_fc1, W_fc2, b_fc2, bn_g, bn_b, W_fb1, b_fb1,
           W_fb2, b_fb2, W_g1, b_g1, W_g2, b_g2, W_qs, W_ks, W_vs, W_qk,
           W_v, b_v, W_t, b_t, abn_g, abn_b):
    idxf, T, x = _stage_a1(features, W_fc1, b_fc1, W_fb1, W_ks, W_vs)
    G = _sc_gather(T.reshape(_B * _N, _TW),
                   idxf.reshape(_B * _N * _K // _CH, _CH))
    q, pd = _stage_a2(features, x, W_fb1, W_qs)
    attnf, out = _stage_cd(G, pd.reshape(_B * _N, _D),
                           q.reshape(_B * _N, _D), x.reshape(_B * _N, _D),
                           W_fb2, b_fb1, b_fb2, W_g1, b_g1, W_g2, b_g2,
                           W_fc2, b_fc2, bn_g, bn_b, W_qk, W_v, b_v, W_t,
                           b_t, abn_g, abn_b)
    return out, attnf


# speedup vs baseline: 1.0013x; 1.0013x over previous
import functools

import jax
import jax.numpy as jnp
from jax import lax
from jax.experimental import pallas as pl
from jax.experimental.pallas import tpu as pltpu
from jax.experimental.pallas import tpu_sc as plsc

_B, _N, _K, _C, _D = 2, 512, 16, 512, 256
_NC, _NS = 2, 16
_NW = _NC * _NS
_TI = 128
_CH = 64
_NB = 3
_TW = 3 * _D // 2

_F32 = jnp.float32
_C11 = (((1,), (1,)), ((), ()))


def _pack_bf16_pair(a, b):
    ua = lax.bitcast_convert_type(a, jnp.uint32)
    ub = lax.bitcast_convert_type(b, jnp.uint32)
    ua = ua + (jnp.uint32(0x7FFF) + ((ua >> 16) & jnp.uint32(1)))
    ub = ub + (jnp.uint32(0x7FFF) + ((ub >> 16) & jnp.uint32(1)))
    packed = (ua & jnp.uint32(0xFFFF0000)) | (ub >> 16)
    return lax.bitcast_convert_type(packed, _F32)


def _unpack_bf16_pair(p):
    up = lax.bitcast_convert_type(p, jnp.uint32)
    a = lax.bitcast_convert_type(up & jnp.uint32(0xFFFF0000), _F32)
    b = lax.bitcast_convert_type(up << 16, _F32)
    return a, b


def _stage_a1_body(f_ref, Wfc1_ref, bfc1_ref, Wfb1_ref, Wks_ref, Wvs_ref,
                   idx_ref, T_ref, x_ref):
    f = f_ref[0]
    m = lax.dot_general(f, f, (((0,), (0,)), ((), ())))
    xx = jnp.sum(f * f, axis=0, keepdims=True)
    inner = -2.0 * m
    p = (-xx) - inner
    iota = lax.broadcasted_iota(jnp.int32, (1, _N), 1)
    cols = []
    for _ in range(_K):
        mx = jnp.max(p, axis=1, keepdims=True)
        sel = jnp.min(jnp.where(p == mx, iota, _N), axis=1, keepdims=True)
        cols.append(sel)
        p = jnp.where(iota == sel, -jnp.inf, p)
    idx = jnp.concatenate(cols, axis=1)
    idx_ref[0] = idx + pl.program_id(0) * _N

    Wfb1 = Wfb1_ref[...]
    Wa = Wfb1[:, :_C]
    cN1 = (((0,), (1,)), ((), ()))
    x = lax.dot_general(f, Wfc1_ref[...], cN1) + bfc1_ref[...]
    pa = lax.dot_general(f, Wa, cN1)
    k_ = lax.dot_general(x, Wks_ref[...], _C11)
    v = lax.dot_general(x, Wvs_ref[...], _C11)
    T = jnp.concatenate([pa, k_, v], axis=1)
    T_ref[0] = _pack_bf16_pair(T[:, :_TW], T[:, _TW:])
    x_ref[0] = x


def _stage_a1(features, W_fc1, b_fc1, W_fb1, W_ks, W_vs, interpret=False):
    def wspec(shape):
        return pl.BlockSpec(shape, lambda b: (0,) * len(shape))

    return pl.pallas_call(
        _stage_a1_body,
        grid=(_B,),
        in_specs=[
            pl.BlockSpec((1, _C, _N), lambda b: (b, 0, 0)),
            wspec((_D, _C)), wspec((1, _D)), wspec((_D, 2 * _C)),
            wspec((_D, _D)), wspec((_D, _D)),
        ],
        out_specs=[
            pl.BlockSpec((1, _N, _K), lambda b: (b, 0, 0)),
            pl.BlockSpec((1, _N, _TW), lambda b: (b, 0, 0)),
            pl.BlockSpec((1, _N, _D), lambda b: (b, 0, 0)),
        ],
        out_shape=[
            jax.ShapeDtypeStruct((_B, _N, _K), jnp.int32),
            jax.ShapeDtypeStruct((_B, _N, _TW), _F32),
            jax.ShapeDtypeStruct((_B, _N, _D), _F32),
        ],
        interpret=interpret,
    )(features, W_fc1, b_fc1.reshape(1, _D), W_fb1, W_ks, W_vs)


def _stage_a2_body(f_ref, x_ref, Wfb1_ref, Wqs_ref, q_ref, pd_ref):
    f = f_ref[0]
    x = x_ref[0]
    Wfb1 = Wfb1_ref[...]
    Wd = Wfb1[:, _C:] - Wfb1[:, :_C]
    cN1 = (((0,), (1,)), ((), ()))
    pd_ref[0] = lax.dot_general(f, Wd, cN1)
    q_ref[0] = lax.dot_general(x, Wqs_ref[...], _C11)


def _stage_a2(features, x, W_fb1, W_qs, interpret=False):
    def wspec(shape):
        return pl.BlockSpec(shape, lambda b: (0,) * len(shape))

    return pl.pallas_call(
        _stage_a2_body,
        grid=(_B,),
        in_specs=[
            pl.BlockSpec((1, _C, _N), lambda b: (b, 0, 0)),
            pl.BlockSpec((1, _N, _D), lambda b: (b, 0, 0)),
            wspec((_D, 2 * _C)), wspec((_D, _D)),
        ],
        out_specs=[
            pl.BlockSpec((1, _N, _D), lambda b: (b, 0, 0)),
            pl.BlockSpec((1, _N, _D), lambda b: (b, 0, 0)),
        ],
        out_shape=[
            jax.ShapeDtypeStruct((_B, _N, _D), _F32),
            jax.ShapeDtypeStruct((_B, _N, _D), _F32),
        ],
        interpret=interpret,
    )(features, x, W_fb1, W_qs)


def _sc_gather(table, idx2d):
    nrows = idx2d.shape[0] * idx2d.shape[1]
    per_w = nrows // _NW
    nch = per_w // _CH
    mesh = plsc.VectorSubcoreMesh(core_axis_name="c", subcore_axis_name="s")

    @functools.partial(
        pl.kernel, mesh=mesh,
        out_type=jax.ShapeDtypeStruct((nrows, _TW), _F32),
        scratch_types=[
            pltpu.VMEM((nch, _CH), jnp.int32),
            pltpu.VMEM((_NB, _CH, _TW), _F32),
        ] + [pltpu.SemaphoreType.DMA] * _NB,
    )
    def gk(table_hbm, idx_hbm, out_hbm, idx_v, rows_v, *sems):
        wid = lax.axis_index("s") * _NC + lax.axis_index("c")
        base = wid * per_w
        pltpu.sync_copy(idx_hbm.at[pl.ds(wid * nch, nch)], idx_v)
        cps = [None] * _NB
        for c in range(min(_NB - 1, nch)):
            cps[c] = pltpu.async_copy(table_hbm.at[idx_v.at[c]],
                                      rows_v.at[c], sems[c])
        for c in range(nch):
            p = c % _NB
            cn = c + _NB - 1
            if cn < nch:
                pn = cn % _NB
                cps[pn] = pltpu.async_copy(table_hbm.at[idx_v.at[cn]],
                                           rows_v.at[pn], sems[pn])
            cps[p].wait()
            pltpu.sync_copy(rows_v.at[p], out_hbm.at[pl.ds(base + c * _CH,
                                                           _CH)])

    return gk(table, idx2d)


def _tail(res1, xr, Wfc2, bfc2, bng, bnb, Wqk, Wv, bv, Wt, bt, abng, abnb):
    y = lax.dot_general(res1, Wfc2, _C11) + bfc2
    m1 = jnp.mean(y, axis=0, keepdims=True)
    v1 = jnp.mean((y - m1) ** 2, axis=0, keepdims=True)
    res = jnp.maximum(bng * (y - m1) * lax.rsqrt(v1 + 1e-5) + bnb, 0.0) + xr
    trs = []
    for b in range(_B):
        rb = res[b * _N:(b + 1) * _N]
        xq = lax.dot_general(rb, Wqk, _C11)
        e = lax.dot_general(xq, xq, _C11)
        ee = jnp.exp(e - jnp.max(e, axis=1, keepdims=True))
        att = ee / jnp.sum(ee, axis=1, keepdims=True)
        att = att / (1e-9 + jnp.sum(att, axis=0, keepdims=True))
        xv = lax.dot_general(rb, Wv, _C11) + bv
        x_r = lax.dot_general(att, xv, (((0,), (0,)), ((), ())))
        trs.append(lax.dot_general(rb - x_r, Wt, _C11) + bt)
    tr = jnp.concatenate(trs, axis=0)
    m2 = jnp.mean(tr, axis=0, keepdims=True)
    v2 = jnp.mean((tr - m2) ** 2, axis=0, keepdims=True)
    xr2 = jnp.maximum(abng * (tr - m2) * lax.rsqrt(v2 + 1e-5) + abnb, 0.0)
    return res + xr2


def _stage_cd_body(G_ref, pd_ref, q_ref, x_ref, Wfb2_ref, bfb1_ref,
                   bfb2_ref, Wg1_ref, bg1_ref, Wg2_ref, bg2_ref, Wfc2_ref,
                   bfc2_ref, bng_ref, bnb_ref, Wqk_ref, Wv_ref, bv_ref,
                   Wt_ref, bt_ref, abng_ref, abnb_ref, attn_ref, out_ref,
                   acc_ref):
    p = pl.program_id(0)
    ghi, glo = _unpack_bf16_pair(G_ref[...])
    G = jnp.concatenate([ghi, glo], axis=1)
    pd = pd_ref[...]
    q = q_ref[...]
    pd_b = jnp.broadcast_to(pd[:, None, :], (_TI, _K, _D)).reshape(_TI * _K,
                                                                   _D)
    q_b = jnp.broadcast_to(q[:, None, :], (_TI, _K, _D)).reshape(_TI * _K, _D)
    h1 = jnp.maximum(G[:, :_D] + pd_b + bfb1_ref[...], 0.0)
    kf = lax.dot_general(h1, Wfb2_ref[...], _C11) + bfb2_ref[...]
    t = q_b - G[:, _D:2 * _D] + kf
    g1 = jnp.maximum(lax.dot_general(t, Wg1_ref[...], _C11) + bg1_ref[...],
                     0.0)
    araw = lax.dot_general(g1, Wg2_ref[...], _C11) + bg2_ref[...]
    s = (araw * (1.0 / 16.0)).reshape(_TI, _K, _D)
    mx = jnp.max(s, axis=1, keepdims=True)
    sh = s - mx
    attn = sh - jnp.log(jnp.sum(jnp.exp(sh), axis=1, keepdims=True))
    attn_ref[...] = attn.reshape(1, _TI, _K, _D)
    vkf = (G[:, 2 * _D:] + kf).reshape(_TI, _K, _D)
    acc_ref[pl.ds(p * _TI, _TI), :] = jnp.sum(attn * vkf, axis=1)

    nt = (_B * _N) // _TI

    @pl.when(p == nt - 1)
    def _():
        out_ref[...] = _tail(
            acc_ref[...], x_ref[...], Wfc2_ref[...], bfc2_ref[...],
            bng_ref[...], bnb_ref[...], Wqk_ref[...], Wv_ref[...],
            bv_ref[...], Wt_ref[...], bt_ref[...], abng_ref[...],
            abnb_ref[...]).reshape(_B, _N, _D)


def _stage_cd(G, pd, q, x, W_fb2, b_fb1, b_fb2, W_g1, b_g1, W_g2, b_g2,
              W_fc2, b_fc2, bn_g, bn_b, W_qk, W_v, b_v, W_t, b_t, abn_g,
              abn_b, interpret=False):
    nt = (_B * _N) // _TI

    def wspec(shape):
        return pl.BlockSpec(shape, lambda p: (0,) * len(shape))

    return pl.pallas_call(
        _stage_cd_body,
        grid=(nt,),
        in_specs=[
            pl.BlockSpec((_TI * _K, _TW), lambda p: (p, 0)),
            pl.BlockSpec((_TI, _D), lambda p: (p, 0)),
            pl.BlockSpec((_TI, _D), lambda p: (p, 0)),
            wspec((_B * _N, _D)),
            wspec((_D, _D)), wspec((1, _D)), wspec((1, _D)),
            wspec((_D, _D)), wspec((1, _D)),
            wspec((_D, _D)), wspec((1, _D)),
            wspec((_D, _D)), wspec((1, _D)), wspec((1, _D)), wspec((1, _D)),
            wspec((_D // 4, _D)),
            wspec((_D, _D)), wspec((1, _D)),
            wspec((_D, _D)), wspec((1, _D)), wspec((1, _D)), wspec((1, _D)),
        ],
        out_specs=[
            pl.BlockSpec((1, _TI, _K, _D),
                         lambda p: (p // (_N // _TI), p % (_N // _TI), 0, 0)),
            wspec((_B, _N, _D)),
        ],
        out_shape=[
            jax.ShapeDtypeStruct((_B, _N, _K, _D), _F32),
            jax.ShapeDtypeStruct((_B, _N, _D), _F32),
        ],
        scratch_shapes=[pltpu.VMEM((_B * _N, _D), _F32)],
        interpret=interpret,
    )(G, pd, q, x, W_fb2, b_fb1.reshape(1, _D), b_fb2.reshape(1, _D),
      W_g1, b_g1.reshape(1, _D), W_g2, b_g2.reshape(1, _D),
      W_fc2, b_fc2.reshape(1, _D), bn_g.reshape(1, _D), bn_b.reshape(1, _D),
      W_qk, W_v, b_v.reshape(1, _D), W_t, b_t.reshape(1, _D),
      abn_g.reshape(1, _D), abn_b.reshape(1, _D))


def kernel(features, W_fc1, b_fc1, W_fc2, b_fc2, bn_g, bn_b, W_fb1, b_fb1,
           W_fb2, b_fb2, W_g1, b_g1, W_g2, b_g2, W_qs, W_ks, W_vs, W_qk,
           W_v, b_v, W_t, b_t, abn_g, abn_b):
    idxf, T, x = _stage_a1(features, W_fc1, b_fc1, W_fb1, W_ks, W_vs)
    G = _sc_gather(T.reshape(_B * _N, _TW),
                   idxf.reshape(_B * _N * _K // _CH, _CH))
    q, pd = _stage_a2(features, x, W_fb1, W_qs)
    attnf, out = _stage_cd(G, pd.reshape(_B * _N, _D),
                           q.reshape(_B * _N, _D), x.reshape(_B * _N, _D),
                           W_fb2, b_fb1, b_fb2, W_g1, b_g1, W_g2, b_g2,
                           W_fc2, b_fc2, bn_g, bn_b, W_qk, W_v, b_v, W_t,
                           b_t, abn_g, abn_b)
    return out, attnf


# speedup vs baseline: 1.0133x; 1.0120x over previous
import functools

import jax
import jax.numpy as jnp
from jax import lax
from jax.experimental import pallas as pl
from jax.experimental.pallas import tpu as pltpu
from jax.experimental.pallas import tpu_sc as plsc

_B, _N, _K, _C, _D = 2, 512, 16, 512, 256
_NC, _NS = 2, 16
_NW = _NC * _NS
_TI = 128
_CH = 64
_NB = 3
_TW = 3 * _D // 2

_F32 = jnp.float32
_C11 = (((1,), (1,)), ((), ()))


def _pack_bf16_pair(a, b):
    ua = lax.bitcast_convert_type(a, jnp.uint32)
    ub = lax.bitcast_convert_type(b, jnp.uint32)
    ua = ua + (jnp.uint32(0x7FFF) + ((ua >> 16) & jnp.uint32(1)))
    ub = ub + (jnp.uint32(0x7FFF) + ((ub >> 16) & jnp.uint32(1)))
    packed = (ua & jnp.uint32(0xFFFF0000)) | (ub >> 16)
    return lax.bitcast_convert_type(packed, _F32)


def _unpack_bf16_pair(p):
    up = lax.bitcast_convert_type(p, jnp.uint32)
    a = lax.bitcast_convert_type(up & jnp.uint32(0xFFFF0000), _F32)
    b = lax.bitcast_convert_type(up << 16, _F32)
    return a, b


def _stage_a1_body(f_ref, Wfc1_ref, bfc1_ref, Wfb1_ref, Wks_ref, Wvs_ref,
                   idx_ref, T_ref, x_ref):
    f = f_ref[0]
    m = lax.dot_general(f, f, (((0,), (0,)), ((), ())))
    xx = jnp.sum(f * f, axis=0, keepdims=True)
    inner = -2.0 * m
    p = (-xx) - inner
    iota = lax.broadcasted_iota(jnp.int32, (1, _N), 1)
    cols = []
    for _ in range(_K):
        mx = jnp.max(p, axis=1, keepdims=True)
        sel = jnp.min(jnp.where(p == mx, iota, _N), axis=1, keepdims=True)
        cols.append(sel)
        p = jnp.where(iota == sel, -jnp.inf, p)
    idx = jnp.concatenate(cols, axis=1)
    idx_ref[0] = idx + pl.program_id(0) * _N

    Wfb1 = Wfb1_ref[...]
    Wa = Wfb1[:, :_C]
    cN1 = (((0,), (1,)), ((), ()))
    x = lax.dot_general(f, Wfc1_ref[...], cN1) + bfc1_ref[...]
    pa = lax.dot_general(f, Wa, cN1)
    k_ = lax.dot_general(x, Wks_ref[...], _C11)
    v = lax.dot_general(x, Wvs_ref[...], _C11)
    T = jnp.concatenate([pa, k_, v], axis=1)
    T_ref[0] = _pack_bf16_pair(T[:, :_TW], T[:, _TW:])
    x_ref[0] = x


def _stage_a1(features, W_fc1, b_fc1, W_fb1, W_ks, W_vs, interpret=False):
    def wspec(shape):
        return pl.BlockSpec(shape, lambda b: (0,) * len(shape))

    return pl.pallas_call(
        _stage_a1_body,
        grid=(_B,),
        in_specs=[
            pl.BlockSpec((1, _C, _N), lambda b: (b, 0, 0)),
            wspec((_D, _C)), wspec((1, _D)), wspec((_D, 2 * _C)),
            wspec((_D, _D)), wspec((_D, _D)),
        ],
        out_specs=[
            pl.BlockSpec((1, _N, _K), lambda b: (b, 0, 0)),
            pl.BlockSpec((1, _N, _TW), lambda b: (b, 0, 0)),
            pl.BlockSpec((1, _N, _D), lambda b: (b, 0, 0)),
        ],
        out_shape=[
            jax.ShapeDtypeStruct((_B, _N, _K), jnp.int32),
            jax.ShapeDtypeStruct((_B, _N, _TW), _F32),
            jax.ShapeDtypeStruct((_B, _N, _D), _F32),
        ],
        interpret=interpret,
    )(features, W_fc1, b_fc1.reshape(1, _D), W_fb1, W_ks, W_vs)


def _stage_a2_body(f_ref, x_ref, Wfb1_ref, Wqs_ref, q_ref, pd_ref):
    f = f_ref[0]
    x = x_ref[0]
    Wfb1 = Wfb1_ref[...]
    Wd = Wfb1[:, _C:] - Wfb1[:, :_C]
    cN1 = (((0,), (1,)), ((), ()))
    pd_ref[0] = lax.dot_general(f, Wd, cN1)
    q_ref[0] = lax.dot_general(x, Wqs_ref[...], _C11)


def _stage_a2(features, x, W_fb1, W_qs, interpret=False):
    def wspec(shape):
        return pl.BlockSpec(shape, lambda b: (0,) * len(shape))

    return pl.pallas_call(
        _stage_a2_body,
        grid=(_B,),
        in_specs=[
            pl.BlockSpec((1, _C, _N), lambda b: (b, 0, 0)),
            pl.BlockSpec((1, _N, _D), lambda b: (b, 0, 0)),
            wspec((_D, 2 * _C)), wspec((_D, _D)),
        ],
        out_specs=[
            pl.BlockSpec((1, _N, _D), lambda b: (b, 0, 0)),
            pl.BlockSpec((1, _N, _D), lambda b: (b, 0, 0)),
        ],
        out_shape=[
            jax.ShapeDtypeStruct((_B, _N, _D), _F32),
            jax.ShapeDtypeStruct((_B, _N, _D), _F32),
        ],
        interpret=interpret,
    )(features, x, W_fb1, W_qs)


def _sc_gather(table, idx2d):
    nrows = idx2d.shape[0] * idx2d.shape[1]
    per_w = nrows // _NW
    nch = per_w // _CH
    mesh = plsc.VectorSubcoreMesh(core_axis_name="c", subcore_axis_name="s")

    @functools.partial(
        pl.kernel, mesh=mesh,
        out_type=jax.ShapeDtypeStruct((nrows, _TW), _F32),
        scratch_types=[
            pltpu.VMEM((nch, _CH), jnp.int32),
            pltpu.VMEM((_NB, _CH, _TW), _F32),
        ] + [pltpu.SemaphoreType.DMA] * _NB,
    )
    def gk(table_hbm, idx_hbm, out_hbm, idx_v, rows_v, *sems):
        wid = lax.axis_index("s") * _NC + lax.axis_index("c")
        base = wid * per_w
        pltpu.sync_copy(idx_hbm.at[pl.ds(wid * nch, nch)], idx_v)
        cps = [None] * _NB
        for c in range(min(_NB - 1, nch)):
            cps[c] = pltpu.async_copy(table_hbm.at[idx_v.at[c]],
                                      rows_v.at[c], sems[c])
        for c in range(nch):
            p = c % _NB
            cn = c + _NB - 1
            if cn < nch:
                pn = cn % _NB
                cps[pn] = pltpu.async_copy(table_hbm.at[idx_v.at[cn]],
                                           rows_v.at[pn], sems[pn])
            cps[p].wait()
            pltpu.sync_copy(rows_v.at[p], out_hbm.at[pl.ds(base + c * _CH,
                                                           _CH)])

    return gk(table, idx2d)


def _tail(res1, xr, Wfc2, bfc2, bng, bnb, Wqk, Wv, bv, Wt, bt, abng, abnb):
    y = lax.dot_general(res1, Wfc2, _C11) + bfc2
    m1 = jnp.mean(y, axis=0, keepdims=True)
    v1 = jnp.mean((y - m1) ** 2, axis=0, keepdims=True)
    res = jnp.maximum(bng * (y - m1) * lax.rsqrt(v1 + 1e-5) + bnb, 0.0) + xr
    trs = []
    for b in range(_B):
        rb = res[b * _N:(b + 1) * _N]
        xq = lax.dot_general(rb, Wqk, _C11)
        e = lax.dot_general(xq, xq, _C11)
        ee = jnp.exp(e - jnp.max(e, axis=1, keepdims=True))
        att = ee / jnp.sum(ee, axis=1, keepdims=True)
        att = att / (1e-9 + jnp.sum(att, axis=0, keepdims=True))
        xv = lax.dot_general(rb, Wv, _C11) + bv
        x_r = lax.dot_general(att, xv, (((0,), (0,)), ((), ())))
        trs.append(lax.dot_general(rb - x_r, Wt, _C11) + bt)
    tr = jnp.concatenate(trs, axis=0)
    m2 = jnp.mean(tr, axis=0, keepdims=True)
    v2 = jnp.mean((tr - m2) ** 2, axis=0, keepdims=True)
    xr2 = jnp.maximum(abng * (tr - m2) * lax.rsqrt(v2 + 1e-5) + abnb, 0.0)
    return res + xr2


def _stage_cd_body(G_ref, pd_ref, q_ref, x_ref, Wfb2_ref, bfb1_ref,
                   bfb2_ref, Wg1_ref, bg1_ref, Wg2_ref, bg2_ref, Wfc2_ref,
                   bfc2_ref, bng_ref, bnb_ref, Wqk_ref, Wv_ref, bv_ref,
                   Wt_ref, bt_ref, abng_ref, abnb_ref, attn_ref, out_ref,
                   acc_ref):
    p = pl.program_id(0)
    ghi, glo = _unpack_bf16_pair(G_ref[...])
    G = jnp.concatenate([ghi, glo], axis=1)
    pd = pd_ref[...]
    q = q_ref[...]
    pd_b = jnp.broadcast_to(pd[:, None, :], (_TI, _K, _D)).reshape(_TI * _K,
                                                                   _D)
    q_b = jnp.broadcast_to(q[:, None, :], (_TI, _K, _D)).reshape(_TI * _K, _D)
    bf = jnp.bfloat16
    h1 = jnp.maximum(G[:, :_D] + pd_b + bfb1_ref[...], 0.0)
    kf = lax.dot_general(h1.astype(bf), Wfb2_ref[...].astype(bf), _C11,
                         preferred_element_type=_F32) + bfb2_ref[...]
    t = q_b - G[:, _D:2 * _D] + kf
    g1 = jnp.maximum(
        lax.dot_general(t.astype(bf), Wg1_ref[...].astype(bf), _C11,
                        preferred_element_type=_F32) + bg1_ref[...], 0.0)
    araw = lax.dot_general(g1.astype(bf), Wg2_ref[...].astype(bf), _C11,
                           preferred_element_type=_F32) + bg2_ref[...]
    s = (araw * (1.0 / 16.0)).reshape(_TI, _K, _D)
    mx = jnp.max(s, axis=1, keepdims=True)
    sh = s - mx
    attn = sh - jnp.log(jnp.sum(jnp.exp(sh), axis=1, keepdims=True))
    attn_ref[...] = attn.reshape(1, _TI, _K, _D)
    vkf = (G[:, 2 * _D:] + kf).reshape(_TI, _K, _D)
    acc_ref[pl.ds(p * _TI, _TI), :] = jnp.sum(attn * vkf, axis=1)

    nt = (_B * _N) // _TI

    @pl.when(p == nt - 1)
    def _():
        out_ref[...] = _tail(
            acc_ref[...], x_ref[...], Wfc2_ref[...], bfc2_ref[...],
            bng_ref[...], bnb_ref[...], Wqk_ref[...], Wv_ref[...],
            bv_ref[...], Wt_ref[...], bt_ref[...], abng_ref[...],
            abnb_ref[...]).reshape(_B, _N, _D)


def _stage_cd(G, pd, q, x, W_fb2, b_fb1, b_fb2, W_g1, b_g1, W_g2, b_g2,
              W_fc2, b_fc2, bn_g, bn_b, W_qk, W_v, b_v, W_t, b_t, abn_g,
              abn_b, interpret=False):
    nt = (_B * _N) // _TI

    def wspec(shape):
        return pl.BlockSpec(shape, lambda p: (0,) * len(shape))

    return pl.pallas_call(
        _stage_cd_body,
        grid=(nt,),
        in_specs=[
            pl.BlockSpec((_TI * _K, _TW), lambda p: (p, 0)),
            pl.BlockSpec((_TI, _D), lambda p: (p, 0)),
            pl.BlockSpec((_TI, _D), lambda p: (p, 0)),
            wspec((_B * _N, _D)),
            wspec((_D, _D)), wspec((1, _D)), wspec((1, _D)),
            wspec((_D, _D)), wspec((1, _D)),
            wspec((_D, _D)), wspec((1, _D)),
            wspec((_D, _D)), wspec((1, _D)), wspec((1, _D)), wspec((1, _D)),
            wspec((_D // 4, _D)),
            wspec((_D, _D)), wspec((1, _D)),
            wspec((_D, _D)), wspec((1, _D)), wspec((1, _D)), wspec((1, _D)),
        ],
        out_specs=[
            pl.BlockSpec((1, _TI, _K, _D),
                         lambda p: (p // (_N // _TI), p % (_N // _TI), 0, 0)),
            wspec((_B, _N, _D)),
        ],
        out_shape=[
            jax.ShapeDtypeStruct((_B, _N, _K, _D), _F32),
            jax.ShapeDtypeStruct((_B, _N, _D), _F32),
        ],
        scratch_shapes=[pltpu.VMEM((_B * _N, _D), _F32)],
        interpret=interpret,
    )(G, pd, q, x, W_fb2, b_fb1.reshape(1, _D), b_fb2.reshape(1, _D),
      W_g1, b_g1.reshape(1, _D), W_g2, b_g2.reshape(1, _D),
      W_fc2, b_fc2.reshape(1, _D), bn_g.reshape(1, _D), bn_b.reshape(1, _D),
      W_qk, W_v, b_v.reshape(1, _D), W_t, b_t.reshape(1, _D),
      abn_g.reshape(1, _D), abn_b.reshape(1, _D))


def kernel(features, W_fc1, b_fc1, W_fc2, b_fc2, bn_g, bn_b, W_fb1, b_fb1,
           W_fb2, b_fb2, W_g1, b_g1, W_g2, b_g2, W_qs, W_ks, W_vs, W_qk,
           W_v, b_v, W_t, b_t, abn_g, abn_b):
    idxf, T, x = _stage_a1(features, W_fc1, b_fc1, W_fb1, W_ks, W_vs)
    G = _sc_gather(T.reshape(_B * _N, _TW),
                   idxf.reshape(_B * _N * _K // _CH, _CH))
    q, pd = _stage_a2(features, x, W_fb1, W_qs)
    attnf, out = _stage_cd(G, pd.reshape(_B * _N, _D),
                           q.reshape(_B * _N, _D), x.reshape(_B * _N, _D),
                           W_fb2, b_fb1, b_fb2, W_g1, b_g1, W_g2, b_g2,
                           W_fc2, b_fc2, bn_g, bn_b, W_qk, W_v, b_v, W_t,
                           b_t, abn_g, abn_b)
    return out, attnf


# speedup vs baseline: 1.0821x; 1.0679x over previous
import functools

import jax
import jax.numpy as jnp
from jax import lax
from jax.experimental import pallas as pl
from jax.experimental.pallas import tpu as pltpu
from jax.experimental.pallas import tpu_sc as plsc

_B, _N, _K, _C, _D = 2, 512, 16, 512, 256
_NC, _NS = 2, 16
_NW = _NC * _NS
_TI = 128
_CH = 64
_NB = 3
_TW = 3 * _D // 2

_F32 = jnp.float32
_C11 = (((1,), (1,)), ((), ()))


def _pack_bf16_pair(a, b):
    ua = lax.bitcast_convert_type(a, jnp.uint32)
    ub = lax.bitcast_convert_type(b, jnp.uint32)
    ua = ua + (jnp.uint32(0x7FFF) + ((ua >> 16) & jnp.uint32(1)))
    ub = ub + (jnp.uint32(0x7FFF) + ((ub >> 16) & jnp.uint32(1)))
    packed = (ua & jnp.uint32(0xFFFF0000)) | (ub >> 16)
    return lax.bitcast_convert_type(packed, _F32)


def _unpack_bf16_pair(p):
    up = lax.bitcast_convert_type(p, jnp.uint32)
    a = lax.bitcast_convert_type(up & jnp.uint32(0xFFFF0000), _F32)
    b = lax.bitcast_convert_type(up << 16, _F32)
    return a, b


def _stage_a1_body(f_ref, Wfc1_ref, bfc1_ref, Wfb1_ref, Wks_ref, Wvs_ref,
                   idx_ref, T_ref, Tbf_ref, x_ref):
    f = f_ref[0]
    m = lax.dot_general(f, f, (((0,), (0,)), ((), ())))
    xx = jnp.sum(f * f, axis=0, keepdims=True)
    inner = -2.0 * m
    p = (-xx) - inner
    iota = lax.broadcasted_iota(jnp.int32, (1, _N), 1)
    cols = []
    for _ in range(_K):
        mx = jnp.max(p, axis=1, keepdims=True)
        sel = jnp.min(jnp.where(p == mx, iota, _N), axis=1, keepdims=True)
        cols.append(sel)
        p = jnp.where(iota == sel, -jnp.inf, p)
    idx = jnp.concatenate(cols, axis=1)
    idx_ref[0] = idx + pl.program_id(0) * _N

    Wfb1 = Wfb1_ref[...]
    Wa = Wfb1[:, :_C]
    cN1 = (((0,), (1,)), ((), ()))
    x = lax.dot_general(f, Wfc1_ref[...], cN1) + bfc1_ref[...]
    pa = lax.dot_general(f, Wa, cN1)
    k_ = lax.dot_general(x, Wks_ref[...], _C11)
    v = lax.dot_general(x, Wvs_ref[...], _C11)
    T = jnp.concatenate([pa, k_, v], axis=1)
    T_ref[0] = _pack_bf16_pair(T[:, :_TW], T[:, _TW:])
    Tbf_ref[0] = T.astype(jnp.bfloat16)
    x_ref[0] = x


def _stage_a1(features, W_fc1, b_fc1, W_fb1, W_ks, W_vs, interpret=False):
    def wspec(shape):
        return pl.BlockSpec(shape, lambda b: (0,) * len(shape))

    return pl.pallas_call(
        _stage_a1_body,
        grid=(_B,),
        in_specs=[
            pl.BlockSpec((1, _C, _N), lambda b: (b, 0, 0)),
            wspec((_D, _C)), wspec((1, _D)), wspec((_D, 2 * _C)),
            wspec((_D, _D)), wspec((_D, _D)),
        ],
        out_specs=[
            pl.BlockSpec((1, _N, _K), lambda b: (b, 0, 0)),
            pl.BlockSpec((1, _N, _TW), lambda b: (b, 0, 0)),
            pl.BlockSpec((1, _N, 3 * _D), lambda b: (b, 0, 0)),
            pl.BlockSpec((1, _N, _D), lambda b: (b, 0, 0)),
        ],
        out_shape=[
            jax.ShapeDtypeStruct((_B, _N, _K), jnp.int32),
            jax.ShapeDtypeStruct((_B, _N, _TW), _F32),
            jax.ShapeDtypeStruct((_B, _N, 3 * _D), jnp.bfloat16),
            jax.ShapeDtypeStruct((_B, _N, _D), _F32),
        ],
        interpret=interpret,
    )(features, W_fc1, b_fc1.reshape(1, _D), W_fb1, W_ks, W_vs)


def _stage_a2_body(f_ref, x_ref, Wfb1_ref, Wqs_ref, q_ref, pd_ref):
    f = f_ref[0]
    x = x_ref[0]
    Wfb1 = Wfb1_ref[...]
    Wd = Wfb1[:, _C:] - Wfb1[:, :_C]
    cN1 = (((0,), (1,)), ((), ()))
    pd_ref[0] = lax.dot_general(f, Wd, cN1)
    q_ref[0] = lax.dot_general(x, Wqs_ref[...], _C11)


def _stage_a2(features, x, W_fb1, W_qs, interpret=False):
    def wspec(shape):
        return pl.BlockSpec(shape, lambda b: (0,) * len(shape))

    return pl.pallas_call(
        _stage_a2_body,
        grid=(_B,),
        in_specs=[
            pl.BlockSpec((1, _C, _N), lambda b: (b, 0, 0)),
            pl.BlockSpec((1, _N, _D), lambda b: (b, 0, 0)),
            wspec((_D, 2 * _C)), wspec((_D, _D)),
        ],
        out_specs=[
            pl.BlockSpec((1, _N, _D), lambda b: (b, 0, 0)),
            pl.BlockSpec((1, _N, _D), lambda b: (b, 0, 0)),
        ],
        out_shape=[
            jax.ShapeDtypeStruct((_B, _N, _D), _F32),
            jax.ShapeDtypeStruct((_B, _N, _D), _F32),
        ],
        interpret=interpret,
    )(features, x, W_fb1, W_qs)


def _sc_gather(table, idx2d):
    nrows = idx2d.shape[0] * idx2d.shape[1]
    per_w = nrows // _NW
    nch = per_w // _CH
    mesh = plsc.VectorSubcoreMesh(core_axis_name="c", subcore_axis_name="s")

    @functools.partial(
        pl.kernel, mesh=mesh,
        out_type=jax.ShapeDtypeStruct((nrows, _TW), _F32),
        scratch_types=[
            pltpu.VMEM((nch, _CH), jnp.int32),
            pltpu.VMEM((_NB, _CH, _TW), _F32),
        ] + [pltpu.SemaphoreType.DMA] * _NB,
    )
    def gk(table_hbm, idx_hbm, out_hbm, idx_v, rows_v, *sems):
        wid = lax.axis_index("s") * _NC + lax.axis_index("c")
        base = wid * per_w
        pltpu.sync_copy(idx_hbm.at[pl.ds(wid * nch, nch)], idx_v)
        cps = [None] * _NB
        for c in range(min(_NB - 1, nch)):
            cps[c] = pltpu.async_copy(table_hbm.at[idx_v.at[c]],
                                      rows_v.at[c], sems[c])
        for c in range(nch):
            p = c % _NB
            cn = c + _NB - 1
            if cn < nch:
                pn = cn % _NB
                cps[pn] = pltpu.async_copy(table_hbm.at[idx_v.at[cn]],
                                           rows_v.at[pn], sems[pn])
            cps[p].wait()
            pltpu.sync_copy(rows_v.at[p], out_hbm.at[pl.ds(base + c * _CH,
                                                           _CH)])

    return gk(table, idx2d)


def _tail(res1, xr, Wfc2, bfc2, bng, bnb, Wqk, Wv, bv, Wt, bt, abng, abnb):
    y = lax.dot_general(res1, Wfc2, _C11) + bfc2
    m1 = jnp.mean(y, axis=0, keepdims=True)
    v1 = jnp.mean((y - m1) ** 2, axis=0, keepdims=True)
    res = jnp.maximum(bng * (y - m1) * lax.rsqrt(v1 + 1e-5) + bnb, 0.0) + xr
    trs = []
    for b in range(_B):
        rb = res[b * _N:(b + 1) * _N]
        xq = lax.dot_general(rb, Wqk, _C11)
        e = lax.dot_general(xq, xq, _C11)
        ee = jnp.exp(e - jnp.max(e, axis=1, keepdims=True))
        att = ee / jnp.sum(ee, axis=1, keepdims=True)
        att = att / (1e-9 + jnp.sum(att, axis=0, keepdims=True))
        xv = lax.dot_general(rb, Wv, _C11) + bv
        x_r = lax.dot_general(att, xv, (((0,), (0,)), ((), ())))
        trs.append(lax.dot_general(rb - x_r, Wt, _C11) + bt)
    tr = jnp.concatenate(trs, axis=0)
    m2 = jnp.mean(tr, axis=0, keepdims=True)
    v2 = jnp.mean((tr - m2) ** 2, axis=0, keepdims=True)
    xr2 = jnp.maximum(abng * (tr - m2) * lax.rsqrt(v2 + 1e-5) + abnb, 0.0)
    return res + xr2


def _edge_mlp(G, p, pd_ref, q_ref, Wfb2_ref, bfb1_ref, bfb2_ref, Wg1_ref,
              bg1_ref, Wg2_ref, bg2_ref, attn_ref, acc_ref):
    pd = pd_ref[...]
    q = q_ref[...]
    pd_b = jnp.broadcast_to(pd[:, None, :], (_TI, _K, _D)).reshape(_TI * _K,
                                                                   _D)
    q_b = jnp.broadcast_to(q[:, None, :], (_TI, _K, _D)).reshape(_TI * _K, _D)
    bf = jnp.bfloat16
    h1 = jnp.maximum(G[:, :_D] + pd_b + bfb1_ref[...], 0.0)
    kf = lax.dot_general(h1.astype(bf), Wfb2_ref[...].astype(bf), _C11,
                         preferred_element_type=_F32) + bfb2_ref[...]
    t = q_b - G[:, _D:2 * _D] + kf
    g1 = jnp.maximum(
        lax.dot_general(t.astype(bf), Wg1_ref[...].astype(bf), _C11,
                        preferred_element_type=_F32) + bg1_ref[...], 0.0)
    araw = lax.dot_general(g1.astype(bf), Wg2_ref[...].astype(bf), _C11,
                           preferred_element_type=_F32) + bg2_ref[...]
    s = (araw * (1.0 / 16.0)).reshape(_TI, _K, _D)
    mx = jnp.max(s, axis=1, keepdims=True)
    sh = s - mx
    attn = sh - jnp.log(jnp.sum(jnp.exp(sh), axis=1, keepdims=True))
    attn_ref[...] = attn.reshape(1, _TI, _K, _D)
    vkf = (G[:, 2 * _D:] + kf).reshape(_TI, _K, _D)
    acc_ref[pl.ds(p * _TI, _TI), :] = jnp.sum(attn * vkf, axis=1)


def _stage_cd_body(G_ref, idx_ref, Tbf_ref, pd_ref, q_ref, x_ref, Wfb2_ref,
                   bfb1_ref, bfb2_ref, Wg1_ref, bg1_ref, Wg2_ref, bg2_ref,
                   Wfc2_ref, bfc2_ref, bng_ref, bnb_ref, Wqk_ref, Wv_ref,
                   bv_ref, Wt_ref, bt_ref, abng_ref, abnb_ref, attn_ref,
                   out_ref, acc_ref):
    p = pl.program_id(0)
    nb = _N // _TI
    nt = (_B * _N) // _TI

    @pl.when(p < nb)
    def _():
        idxb = idx_ref[0]
        oh = (idxb[:, :, None] ==
              lax.broadcasted_iota(jnp.int32, (1, 1, _N), 2))
        oh2 = oh.astype(jnp.bfloat16).reshape(_TI * _K, _N)
        Goh = lax.dot_general(oh2, Tbf_ref[0], (((1,), (0,)), ((), ())),
                              preferred_element_type=_F32)
        _edge_mlp(Goh, p, pd_ref, q_ref, Wfb2_ref, bfb1_ref, bfb2_ref,
                  Wg1_ref, bg1_ref, Wg2_ref, bg2_ref, attn_ref, acc_ref)

    @pl.when(p >= nb)
    def _():
        ghi, glo = _unpack_bf16_pair(G_ref[...])
        Gsc = jnp.concatenate([ghi, glo], axis=1)
        _edge_mlp(Gsc, p, pd_ref, q_ref, Wfb2_ref, bfb1_ref, bfb2_ref,
                  Wg1_ref, bg1_ref, Wg2_ref, bg2_ref, attn_ref, acc_ref)

    @pl.when(p == nt - 1)
    def _():
        out_ref[...] = _tail(
            acc_ref[...], x_ref[...], Wfc2_ref[...], bfc2_ref[...],
            bng_ref[...], bnb_ref[...], Wqk_ref[...], Wv_ref[...],
            bv_ref[...], Wt_ref[...], bt_ref[...], abng_ref[...],
            abnb_ref[...]).reshape(_B, _N, _D)


def _stage_cd(G, idx, Tbf, pd, q, x, W_fb2, b_fb1, b_fb2, W_g1, b_g1, W_g2,
              b_g2, W_fc2, b_fc2, bn_g, bn_b, W_qk, W_v, b_v, W_t, b_t,
              abn_g, abn_b, interpret=False):
    nt = (_B * _N) // _TI
    nb = _N // _TI

    def wspec(shape):
        return pl.BlockSpec(shape, lambda p: (0,) * len(shape))

    return pl.pallas_call(
        _stage_cd_body,
        grid=(nt,),
        in_specs=[
            pl.BlockSpec((_TI * _K, _TW),
                         lambda p: (jnp.maximum(p - nb, 0), 0)),
            pl.BlockSpec((1, _TI, _K),
                         lambda p: (0, jnp.minimum(p, nb - 1), 0)),
            pl.BlockSpec((1, _N, 3 * _D), lambda p: (0, 0, 0)),
            pl.BlockSpec((_TI, _D), lambda p: (p, 0)),
            pl.BlockSpec((_TI, _D), lambda p: (p, 0)),
            wspec((_B * _N, _D)),
            wspec((_D, _D)), wspec((1, _D)), wspec((1, _D)),
            wspec((_D, _D)), wspec((1, _D)),
            wspec((_D, _D)), wspec((1, _D)),
            wspec((_D, _D)), wspec((1, _D)), wspec((1, _D)), wspec((1, _D)),
            wspec((_D // 4, _D)),
            wspec((_D, _D)), wspec((1, _D)),
            wspec((_D, _D)), wspec((1, _D)), wspec((1, _D)), wspec((1, _D)),
        ],
        out_specs=[
            pl.BlockSpec((1, _TI, _K, _D),
                         lambda p: (p // (_N // _TI), p % (_N // _TI), 0, 0)),
            wspec((_B, _N, _D)),
        ],
        out_shape=[
            jax.ShapeDtypeStruct((_B, _N, _K, _D), _F32),
            jax.ShapeDtypeStruct((_B, _N, _D), _F32),
        ],
        scratch_shapes=[pltpu.VMEM((_B * _N, _D), _F32)],
        interpret=interpret,
    )(G, idx, Tbf, pd, q, x, W_fb2, b_fb1.reshape(1, _D),
      b_fb2.reshape(1, _D),
      W_g1, b_g1.reshape(1, _D), W_g2, b_g2.reshape(1, _D),
      W_fc2, b_fc2.reshape(1, _D), bn_g.reshape(1, _D), bn_b.reshape(1, _D),
      W_qk, W_v, b_v.reshape(1, _D), W_t, b_t.reshape(1, _D),
      abn_g.reshape(1, _D), abn_b.reshape(1, _D))


def kernel(features, W_fc1, b_fc1, W_fc2, b_fc2, bn_g, bn_b, W_fb1, b_fb1,
           W_fb2, b_fb2, W_g1, b_g1, W_g2, b_g2, W_qs, W_ks, W_vs, W_qk,
           W_v, b_v, W_t, b_t, abn_g, abn_b):
    idxf, T, Tbf, x = _stage_a1(features, W_fc1, b_fc1, W_fb1, W_ks, W_vs)
    G1 = _sc_gather(T.reshape(_B * _N, _TW),
                    idxf[1].reshape(_N * _K // _CH, _CH))
    q, pd = _stage_a2(features, x, W_fb1, W_qs)
    attnf, out = _stage_cd(G1, idxf, Tbf, pd.reshape(_B * _N, _D),
                           q.reshape(_B * _N, _D), x.reshape(_B * _N, _D),
                           W_fb2, b_fb1, b_fb2, W_g1, b_g1, W_g2, b_g2,
                           W_fc2, b_fc2, bn_g, bn_b, W_qk, W_v, b_v, W_t,
                           b_t, abn_g, abn_b)
    return out, attnf


# speedup vs baseline: 1.2904x; 1.1925x over previous
import functools

import jax
import jax.numpy as jnp
from jax import lax
from jax.experimental import pallas as pl
from jax.experimental.pallas import tpu as pltpu
from jax.experimental.pallas import tpu_sc as plsc

_B, _N, _K, _C, _D = 2, 512, 16, 512, 256
_NC, _NS = 2, 16
_NW = _NC * _NS
_TI = 128
_CH = 64
_NB = 3
_TW = 3 * _D // 2

_F32 = jnp.float32
_C11 = (((1,), (1,)), ((), ()))


def _pack_bf16_pair(a, b):
    ua = lax.bitcast_convert_type(a, jnp.uint32)
    ub = lax.bitcast_convert_type(b, jnp.uint32)
    ua = ua + (jnp.uint32(0x7FFF) + ((ua >> 16) & jnp.uint32(1)))
    ub = ub + (jnp.uint32(0x7FFF) + ((ub >> 16) & jnp.uint32(1)))
    packed = (ua & jnp.uint32(0xFFFF0000)) | (ub >> 16)
    return lax.bitcast_convert_type(packed, _F32)


def _unpack_bf16_pair(p):
    up = lax.bitcast_convert_type(p, jnp.uint32)
    a = lax.bitcast_convert_type(up & jnp.uint32(0xFFFF0000), _F32)
    b = lax.bitcast_convert_type(up << 16, _F32)
    return a, b


def _stage_a1_body(f_ref, Wfc1_ref, bfc1_ref, Wfb1_ref, Wks_ref, Wvs_ref,
                   idx_ref, T_ref, Tbf_ref, x_ref):
    f = f_ref[0]
    m = lax.dot_general(f, f, (((0,), (0,)), ((), ())))
    xx = jnp.sum(f * f, axis=0, keepdims=True)
    inner = -2.0 * m
    p = (-xx) - inner
    iota = lax.broadcasted_iota(jnp.int32, (1, _N), 1)
    cols = []
    for _ in range(_K):
        mx = jnp.max(p, axis=1, keepdims=True)
        sel = jnp.min(jnp.where(p == mx, iota, _N), axis=1, keepdims=True)
        cols.append(sel)
        p = jnp.where(iota == sel, -jnp.inf, p)
    idx = jnp.concatenate(cols, axis=1)
    idx_ref[0] = idx + pl.program_id(0) * _N

    Wfb1 = Wfb1_ref[...]
    Wa = Wfb1[:, :_C]
    cN1 = (((0,), (1,)), ((), ()))
    x = lax.dot_general(f, Wfc1_ref[...], cN1) + bfc1_ref[...]
    pa = lax.dot_general(f, Wa, cN1)
    k_ = lax.dot_general(x, Wks_ref[...], _C11)
    v = lax.dot_general(x, Wvs_ref[...], _C11)
    T = jnp.concatenate([pa, k_, v], axis=1)
    T_ref[0] = _pack_bf16_pair(T[:, :_TW], T[:, _TW:])
    Tbf_ref[0] = T.astype(jnp.bfloat16)
    x_ref[0] = x


def _stage_a1(features, W_fc1, b_fc1, W_fb1, W_ks, W_vs, interpret=False):
    def wspec(shape):
        return pl.BlockSpec(shape, lambda b: (0,) * len(shape))

    return pl.pallas_call(
        _stage_a1_body,
        grid=(_B,),
        in_specs=[
            pl.BlockSpec((1, _C, _N), lambda b: (b, 0, 0)),
            wspec((_D, _C)), wspec((1, _D)), wspec((_D, 2 * _C)),
            wspec((_D, _D)), wspec((_D, _D)),
        ],
        out_specs=[
            pl.BlockSpec((1, _N, _K), lambda b: (b, 0, 0)),
            pl.BlockSpec((1, _N, _TW), lambda b: (b, 0, 0)),
            pl.BlockSpec((1, _N, 3 * _D), lambda b: (b, 0, 0)),
            pl.BlockSpec((1, _N, _D), lambda b: (b, 0, 0)),
        ],
        out_shape=[
            jax.ShapeDtypeStruct((_B, _N, _K), jnp.int32),
            jax.ShapeDtypeStruct((_B, _N, _TW), _F32),
            jax.ShapeDtypeStruct((_B, _N, 3 * _D), jnp.bfloat16),
            jax.ShapeDtypeStruct((_B, _N, _D), _F32),
        ],
        interpret=interpret,
    )(features, W_fc1, b_fc1.reshape(1, _D), W_fb1, W_ks, W_vs)


def _stage_a2_body(f_ref, x_ref, Wfb1_ref, Wqs_ref, q_ref, pd_ref):
    f = f_ref[0]
    x = x_ref[0]
    Wfb1 = Wfb1_ref[...]
    Wd = Wfb1[:, _C:] - Wfb1[:, :_C]
    cN1 = (((0,), (1,)), ((), ()))
    pd_ref[0] = lax.dot_general(f, Wd, cN1)
    q_ref[0] = lax.dot_general(x, Wqs_ref[...], _C11)


def _stage_a2(features, x, W_fb1, W_qs, interpret=False):
    def wspec(shape):
        return pl.BlockSpec(shape, lambda b: (0,) * len(shape))

    return pl.pallas_call(
        _stage_a2_body,
        grid=(_B,),
        in_specs=[
            pl.BlockSpec((1, _C, _N), lambda b: (b, 0, 0)),
            pl.BlockSpec((1, _N, _D), lambda b: (b, 0, 0)),
            wspec((_D, 2 * _C)), wspec((_D, _D)),
        ],
        out_specs=[
            pl.BlockSpec((1, _N, _D), lambda b: (b, 0, 0)),
            pl.BlockSpec((1, _N, _D), lambda b: (b, 0, 0)),
        ],
        out_shape=[
            jax.ShapeDtypeStruct((_B, _N, _D), _F32),
            jax.ShapeDtypeStruct((_B, _N, _D), _F32),
        ],
        interpret=interpret,
    )(features, x, W_fb1, W_qs)


def _sc_gather(table, idx2d):
    nrows = idx2d.shape[0] * idx2d.shape[1]
    per_w = nrows // _NW
    nch = per_w // _CH
    mesh = plsc.VectorSubcoreMesh(core_axis_name="c", subcore_axis_name="s")

    @functools.partial(
        pl.kernel, mesh=mesh,
        out_type=jax.ShapeDtypeStruct((nrows, _TW), _F32),
        scratch_types=[
            pltpu.VMEM((nch, _CH), jnp.int32),
            pltpu.VMEM((_NB, _CH, _TW), _F32),
        ] + [pltpu.SemaphoreType.DMA] * _NB,
    )
    def gk(table_hbm, idx_hbm, out_hbm, idx_v, rows_v, *sems):
        wid = lax.axis_index("s") * _NC + lax.axis_index("c")
        base = wid * per_w
        pltpu.sync_copy(idx_hbm.at[pl.ds(wid * nch, nch)], idx_v)
        cps = [None] * _NB
        for c in range(min(_NB - 1, nch)):
            cps[c] = pltpu.async_copy(table_hbm.at[idx_v.at[c]],
                                      rows_v.at[c], sems[c])
        for c in range(nch):
            p = c % _NB
            cn = c + _NB - 1
            if cn < nch:
                pn = cn % _NB
                cps[pn] = pltpu.async_copy(table_hbm.at[idx_v.at[cn]],
                                           rows_v.at[pn], sems[pn])
            cps[p].wait()
            pltpu.sync_copy(rows_v.at[p], out_hbm.at[pl.ds(base + c * _CH,
                                                           _CH)])

    return gk(table, idx2d)


def _tail(res1, xr, Wfc2, bfc2, bng, bnb, Wqk, Wv, bv, Wt, bt, abng, abnb):
    y = lax.dot_general(res1, Wfc2, _C11) + bfc2
    m1 = jnp.mean(y, axis=0, keepdims=True)
    v1 = jnp.mean((y - m1) ** 2, axis=0, keepdims=True)
    res = jnp.maximum(bng * (y - m1) * lax.rsqrt(v1 + 1e-5) + bnb, 0.0) + xr
    trs = []
    for b in range(_B):
        rb = res[b * _N:(b + 1) * _N]
        xq = lax.dot_general(rb, Wqk, _C11)
        e = lax.dot_general(xq, xq, _C11)
        ee = jnp.exp(e - jnp.max(e, axis=1, keepdims=True))
        att = ee / jnp.sum(ee, axis=1, keepdims=True)
        att = att / (1e-9 + jnp.sum(att, axis=0, keepdims=True))
        xv = lax.dot_general(rb, Wv, _C11) + bv
        x_r = lax.dot_general(att, xv, (((0,), (0,)), ((), ())))
        trs.append(lax.dot_general(rb - x_r, Wt, _C11) + bt)
    tr = jnp.concatenate(trs, axis=0)
    m2 = jnp.mean(tr, axis=0, keepdims=True)
    v2 = jnp.mean((tr - m2) ** 2, axis=0, keepdims=True)
    xr2 = jnp.maximum(abng * (tr - m2) * lax.rsqrt(v2 + 1e-5) + abnb, 0.0)
    return res + xr2


def _edge_mlp(G, pd_ref, q_ref, Wfb2_ref, bfb1_ref, bfb2_ref, Wg1_ref,
              bg1_ref, Wg2_ref, bg2_ref):
    pd = pd_ref[...]
    q = q_ref[...]
    pd_b = jnp.broadcast_to(pd[:, None, :], (_TI, _K, _D)).reshape(_TI * _K,
                                                                   _D)
    q_b = jnp.broadcast_to(q[:, None, :], (_TI, _K, _D)).reshape(_TI * _K, _D)
    bf = jnp.bfloat16
    h1 = jnp.maximum(G[:, :_D] + pd_b + bfb1_ref[...], 0.0)
    kf = lax.dot_general(h1.astype(bf), Wfb2_ref[...].astype(bf), _C11,
                         preferred_element_type=_F32) + bfb2_ref[...]
    t = q_b - G[:, _D:2 * _D] + kf
    g1 = jnp.maximum(
        lax.dot_general(t.astype(bf), Wg1_ref[...].astype(bf), _C11,
                        preferred_element_type=_F32) + bg1_ref[...], 0.0)
    araw = lax.dot_general(g1.astype(bf), Wg2_ref[...].astype(bf), _C11,
                           preferred_element_type=_F32) + bg2_ref[...]
    s = (araw * (1.0 / 16.0)).reshape(_TI, _K, _D)
    mx = jnp.max(s, axis=1, keepdims=True)
    sh = s - mx
    attn = sh - jnp.log(jnp.sum(jnp.exp(sh), axis=1, keepdims=True))
    vkf = (G[:, 2 * _D:] + kf).reshape(_TI, _K, _D)
    return attn.reshape(1, _TI, _K, _D), jnp.sum(attn * vkf, axis=1)


def _stage_c0_body(idx_ref, Tbf_ref, pd_ref, q_ref, Wfb2_ref, bfb1_ref,
                   bfb2_ref, Wg1_ref, bg1_ref, Wg2_ref, bg2_ref, attn_ref,
                   res1_ref):
    idxb = idx_ref[0]
    oh = (idxb[:, :, None] ==
          lax.broadcasted_iota(jnp.int32, (1, 1, _N), 2))
    oh2 = oh.astype(jnp.bfloat16).reshape(_TI * _K, _N)
    Goh = lax.dot_general(oh2, Tbf_ref[0], (((1,), (0,)), ((), ())),
                          preferred_element_type=_F32)
    attn4, res1 = _edge_mlp(Goh, pd_ref, q_ref, Wfb2_ref, bfb1_ref,
                            bfb2_ref, Wg1_ref, bg1_ref, Wg2_ref, bg2_ref)
    attn_ref[...] = attn4
    res1_ref[...] = res1


def _stage_c1_body(attn_in_ref, G_ref, pd_ref, q_ref, x_ref, res10_ref,
                   Wfb2_ref, bfb1_ref, bfb2_ref, Wg1_ref, bg1_ref, Wg2_ref,
                   bg2_ref, Wfc2_ref, bfc2_ref, bng_ref, bnb_ref, Wqk_ref,
                   Wv_ref, bv_ref, Wt_ref, bt_ref, abng_ref, abnb_ref,
                   attn_ref, out_ref, acc_ref):
    del attn_in_ref
    p = pl.program_id(0)
    nb = _N // _TI
    ghi, glo = _unpack_bf16_pair(G_ref[...])
    Gsc = jnp.concatenate([ghi, glo], axis=1)
    attn4, res1 = _edge_mlp(Gsc, pd_ref, q_ref, Wfb2_ref, bfb1_ref,
                            bfb2_ref, Wg1_ref, bg1_ref, Wg2_ref, bg2_ref)
    attn_ref[...] = attn4
    acc_ref[pl.ds(p * _TI, _TI), :] = res1

    @pl.when(p == nb - 1)
    def _():
        r1 = jnp.concatenate([res10_ref[...], acc_ref[...]], axis=0)
        out_ref[...] = _tail(
            r1, x_ref[...], Wfc2_ref[...], bfc2_ref[...],
            bng_ref[...], bnb_ref[...], Wqk_ref[...], Wv_ref[...],
            bv_ref[...], Wt_ref[...], bt_ref[...], abng_ref[...],
            abnb_ref[...]).reshape(_B, _N, _D)


def _stage_c0(idx, Tbf, pd, q, W_fb2, b_fb1, b_fb2, W_g1, b_g1, W_g2, b_g2,
              interpret=False):
    nb = _N // _TI

    def wspec(shape):
        return pl.BlockSpec(shape, lambda p: (0,) * len(shape))

    return pl.pallas_call(
        _stage_c0_body,
        grid=(nb,),
        in_specs=[
            pl.BlockSpec((1, _TI, _K), lambda p: (0, p, 0)),
            pl.BlockSpec((1, _N, 3 * _D), lambda p: (0, 0, 0)),
            pl.BlockSpec((_TI, _D), lambda p: (p, 0)),
            pl.BlockSpec((_TI, _D), lambda p: (p, 0)),
            wspec((_D, _D)), wspec((1, _D)), wspec((1, _D)),
            wspec((_D, _D)), wspec((1, _D)),
            wspec((_D, _D)), wspec((1, _D)),
        ],
        out_specs=[
            pl.BlockSpec((1, _TI, _K, _D), lambda p: (0, p, 0, 0)),
            pl.BlockSpec((_TI, _D), lambda p: (p, 0)),
        ],
        out_shape=[
            jax.ShapeDtypeStruct((_B, _N, _K, _D), _F32),
            jax.ShapeDtypeStruct((_N, _D), _F32),
        ],
        interpret=interpret,
    )(idx, Tbf, pd, q, W_fb2, b_fb1.reshape(1, _D), b_fb2.reshape(1, _D),
      W_g1, b_g1.reshape(1, _D), W_g2, b_g2.reshape(1, _D))


def _stage_c1(attn0, G, pd, q, x, res10, W_fb2, b_fb1, b_fb2, W_g1, b_g1,
              W_g2, b_g2, W_fc2, b_fc2, bn_g, bn_b, W_qk, W_v, b_v, W_t,
              b_t, abn_g, abn_b, interpret=False):
    nb = _N // _TI

    def wspec(shape):
        return pl.BlockSpec(shape, lambda p: (0,) * len(shape))

    return pl.pallas_call(
        _stage_c1_body,
        grid=(nb,),
        in_specs=[
            pl.BlockSpec(memory_space=pl.ANY),
            pl.BlockSpec((_TI * _K, _TW), lambda p: (p, 0)),
            pl.BlockSpec((_TI, _D), lambda p: (p + nb, 0)),
            pl.BlockSpec((_TI, _D), lambda p: (p + nb, 0)),
            wspec((_B * _N, _D)),
            wspec((_N, _D)),
            wspec((_D, _D)), wspec((1, _D)), wspec((1, _D)),
            wspec((_D, _D)), wspec((1, _D)),
            wspec((_D, _D)), wspec((1, _D)),
            wspec((_D, _D)), wspec((1, _D)), wspec((1, _D)), wspec((1, _D)),
            wspec((_D // 4, _D)),
            wspec((_D, _D)), wspec((1, _D)),
            wspec((_D, _D)), wspec((1, _D)), wspec((1, _D)), wspec((1, _D)),
        ],
        out_specs=[
            pl.BlockSpec((1, _TI, _K, _D), lambda p: (1, p, 0, 0)),
            wspec((_B, _N, _D)),
        ],
        out_shape=[
            jax.ShapeDtypeStruct((_B, _N, _K, _D), _F32),
            jax.ShapeDtypeStruct((_B, _N, _D), _F32),
        ],
        scratch_shapes=[pltpu.VMEM((_N, _D), _F32)],
        input_output_aliases={0: 0},
        interpret=interpret,
    )(attn0, G, pd, q, x, res10, W_fb2, b_fb1.reshape(1, _D),
      b_fb2.reshape(1, _D),
      W_g1, b_g1.reshape(1, _D), W_g2, b_g2.reshape(1, _D),
      W_fc2, b_fc2.reshape(1, _D), bn_g.reshape(1, _D), bn_b.reshape(1, _D),
      W_qk, W_v, b_v.reshape(1, _D), W_t, b_t.reshape(1, _D),
      abn_g.reshape(1, _D), abn_b.reshape(1, _D))


def kernel(features, W_fc1, b_fc1, W_fc2, b_fc2, bn_g, bn_b, W_fb1, b_fb1,
           W_fb2, b_fb2, W_g1, b_g1, W_g2, b_g2, W_qs, W_ks, W_vs, W_qk,
           W_v, b_v, W_t, b_t, abn_g, abn_b):
    idxf, T, Tbf, x = _stage_a1(features, W_fc1, b_fc1, W_fb1, W_ks, W_vs)
    G1 = _sc_gather(T.reshape(_B * _N, _TW),
                    idxf[1].reshape(_N * _K // _CH, _CH))
    q, pd = _stage_a2(features, x, W_fb1, W_qs)
    pdf = pd.reshape(_B * _N, _D)
    qf = q.reshape(_B * _N, _D)
    attn0, res10 = _stage_c0(idxf, Tbf, pdf, qf, W_fb2, b_fb1, b_fb2,
                             W_g1, b_g1, W_g2, b_g2)
    attnf, out = _stage_c1(attn0, G1, pdf, qf, x.reshape(_B * _N, _D),
                           res10, W_fb2, b_fb1, b_fb2, W_g1, b_g1, W_g2,
                           b_g2, W_fc2, b_fc2, bn_g, bn_b, W_qk, W_v, b_v,
                           W_t, b_t, abn_g, abn_b)
    return out, attnf
